# baseline (speedup 1.0000x reference)
import functools
import math

import jax
import jax.numpy as jnp
from jax import lax
from jax.experimental import pallas as pl
from jax.experimental.pallas import tpu as pltpu
from jax.experimental.pallas import tpu_sc as plsc

D_MODEL = 32
SCALE = math.sqrt(float(D_MODEL))
NC = 2
NS = 16
NW = NC * NS
C = 128
NBUF = 8


def _body(nchunks, x_hbm, lut_hbm, out_hbm, *bufs):
  rows_in = bufs[0:NBUF]
  rows_out = bufs[NBUF:2 * NBUF]
  idx_c = bufs[2 * NBUF:3 * NBUF]
  gsem = bufs[3 * NBUF:4 * NBUF]
  osem = bufs[4 * NBUF:5 * NBUF]
  isem = bufs[5 * NBUF:6 * NBUF]

  n = nchunks * C
  wid = lax.axis_index("s") * NC + lax.axis_index("c")
  base = pl.multiple_of(wid * n, 8)

  def idx_copy(chunk, b, start_fn):
    return start_fn(
        x_hbm.at[pl.ds(base + chunk * C, C)], idx_c[b], isem[b])

  def gather(b, start_fn):
    return start_fn(lut_hbm.at[idx_c[b]], rows_in[b], gsem[b])

  def out_copy(chunk, b, start_fn):
    return start_fn(
        rows_out[b], out_hbm.at[pl.ds(base + chunk * C, C)], osem[b])

  for b in range(NBUF):
    pltpu.sync_copy(x_hbm.at[pl.ds(base + b * C, C)], idx_c[b])
    gather(b, pltpu.async_copy)

  nrounds = nchunks // NBUF

  @pl.loop(0, nrounds)
  def _(r):
    for b in range(NBUF):
      chunk = r * NBUF + b
      gather(b, pltpu.make_async_copy).wait()

      @pl.when(r > 0)
      def _():
        out_copy(chunk - NBUF, b, pltpu.make_async_copy).wait()

      @pl.when(r < nrounds - 1)
      def _():
        idx_copy(chunk + NBUF, b, pltpu.async_copy)

      @pl.loop(0, C, unroll=8)
      def _(row):
        for h in (0, 16):
          rows_out[b][row, pl.ds(h, 16)] = (
              rows_in[b][row, pl.ds(h, 16)] * SCALE)

      @pl.when(r < nrounds - 1)
      def _():
        idx_copy(chunk + NBUF, b, pltpu.make_async_copy).wait()
        gather(b, pltpu.async_copy)

      out_copy(chunk, b, pltpu.async_copy)

  for b in range(NBUF):
    out_copy(nchunks - NBUF + b, b, pltpu.make_async_copy).wait()


def kernel(x, lut):
  b_total = x.shape[0] * x.shape[1]
  n = b_total // NW
  nchunks = n // C
  assert n % C == 0 and nchunks % NBUF == 0

  xf = x.reshape(b_total)
  mesh = plsc.VectorSubcoreMesh(
      core_axis_name="c", subcore_axis_name="s",
      num_cores=NC, num_subcores=NS)
  scratch = (
      [pltpu.VMEM((C, D_MODEL), jnp.float32) for _ in range(2 * NBUF)]
      + [pltpu.VMEM((C,), jnp.int32) for _ in range(NBUF)]
      + [pltpu.SemaphoreType.DMA for _ in range(3 * NBUF)]
  )
  out = pl.kernel(
      functools.partial(_body, nchunks),
      out_type=jax.ShapeDtypeStruct((b_total, D_MODEL), jnp.float32),
      mesh=mesh,
      scratch_types=scratch,
      compiler_params=pltpu.CompilerParams(use_tc_tiling_on_sc=False),
  )(xf, lut)
  return out.reshape(x.shape[0], x.shape[1], D_MODEL)
